# baseline (speedup 1.0000x reference)
import functools
import math

import jax
import jax.numpy as jnp
from jax.experimental import pallas as pl

B = 4096
D = 64
_EPS = 1e-5


def _pool_kernel(t_ref, i_ref, p_ref):
    t = t_ref[...] * (1.0 / math.sqrt(D))
    im = i_ref[...]
    s = t[:, :, None] * im[:, None, :]
    m = jnp.max(s, axis=-1, keepdims=True)
    e = jnp.exp(s - m)
    acc = e[..., 0:8]
    for tt in range(1, 8):
        acc = acc + e[..., 8 * tt:8 * tt + 8]
    a4 = acc[..., 0:4] + acc[..., 4:8]
    a2 = a4[..., 0:2] + a4[..., 2:4]
    denom = a2[..., 0] + a2[..., 1]
    p_ref[...] = 1.0 / denom


def _batch_sum(y):
    y4 = y.reshape(32, 16, 8, D)
    acc = y4[0]
    for mm in range(1, 32):
        acc = acc + y4[mm]
    c = acc[0]
    for u in range(1, 16):
        c = c + acc[u]
    c4 = c[0:4] + c[4:8]
    c2 = c4[0:2] + c4[2:4]
    return c2[0:1] + c2[1:2]


def _bnlin_kernel(x_ref, wt_ref, b_ref, g_ref, be_ref, o_ref):
    x = x_ref[...].astype(jnp.bfloat16)
    y = jnp.dot(x, wt_ref[...].astype(jnp.bfloat16),
                preferred_element_type=jnp.float32) + b_ref[...]
    m = _batch_sum(y) * (1.0 / B)
    d = y - m
    v = _batch_sum(d * d) * (1.0 / B)
    o_ref[...] = jnp.maximum(
        d / jnp.sqrt(v + _EPS) * g_ref[...] + be_ref[...], 0.0)


def _mmT_kernel(hT_ref, a_ref, o_ref):
    o_ref[...] = jax.lax.dot_general(
        hT_ref[...].astype(jnp.bfloat16), a_ref[...].astype(jnp.bfloat16),
        (((1,), (1,)), ((), ())), preferred_element_type=jnp.float32)


@functools.partial(jax.jit)
def _run(text, image, in_aff, out_aff, Wc_t, b_c, g_c, be_c,
         W1_t, b1, g1, be1, W2_t, b2, g2, be2):
    BA = 64
    pool = pl.pallas_call(
        _pool_kernel,
        grid=(B // BA,),
        in_specs=[
            pl.BlockSpec((BA, D), lambda i: (i, 0)),
            pl.BlockSpec((BA, D), lambda i: (i, 0)),
        ],
        out_specs=pl.BlockSpec((BA, D), lambda i: (i, 0)),
        out_shape=jax.ShapeDtypeStruct((B, D), jnp.float32),
    )
    p = pool(text, image)

    bnlin = pl.pallas_call(
        _bnlin_kernel,
        out_shape=jax.ShapeDtypeStruct((B, D), jnp.float32),
    )

    BN = 512
    mm = pl.pallas_call(
        _mmT_kernel,
        grid=(B // BN,),
        in_specs=[
            pl.BlockSpec((D, B), lambda i: (0, 0)),
            pl.BlockSpec((BN, B), lambda i: (i, 0)),
        ],
        out_specs=pl.BlockSpec((D, BN), lambda i: (0, i)),
        out_shape=jax.ShapeDtypeStruct((D, B), jnp.float32),
    )

    h0 = bnlin(p, Wc_t, b_c, g_c, be_c)
    a1 = mm(h0.T, in_aff).T
    h1 = bnlin(a1, W1_t, b1, g1, be1)
    a2 = mm(h1.T, out_aff).T
    h2 = bnlin(a2, W2_t, b2, g2, be2)
    return h2


def kernel(text, image, in_aff, out_aff, W_c, b_c, g_c, be_c,
           W1, b1, g1, be1, W2, b2, g2, be2):
    r = lambda x: x.reshape(1, D)
    return _run(text, image, in_aff, out_aff,
                W_c.T, r(b_c), r(g_c), r(be_c),
                W1.T, r(b1), r(g1), r(be1),
                W2.T, r(b2), r(g2), r(be2))


# speedup vs baseline: 5.4563x; 5.4563x over previous
import functools
import math

import jax
import jax.numpy as jnp
from jax.experimental import pallas as pl

B = 4096
D = 64
_EPS = 1e-5


def _pool_kernel(tT_ref, iT_ref, pT_ref):
    tT = tT_ref[...] * (1.0 / math.sqrt(D))
    iT = iT_ref[...]
    s = tT[:, None, :] * iT[None, :, :]
    m = jnp.max(s, axis=1, keepdims=True)
    e = jnp.exp(s - m)
    acc = e[:, 0:8, :]
    for tt in range(1, 8):
        acc = acc + e[:, 8 * tt:8 * tt + 8, :]
    a4 = acc[:, 0:4, :] + acc[:, 4:8, :]
    a2 = a4[:, 0:2, :] + a4[:, 2:4, :]
    denom = a2[:, 0, :] + a2[:, 1, :]
    pT_ref[...] = 1.0 / denom


def _batch_sum(y):
    y4 = y.reshape(32, 16, 8, D)
    acc = y4[0]
    for mm in range(1, 32):
        acc = acc + y4[mm]
    c = acc[0]
    for u in range(1, 16):
        c = c + acc[u]
    c4 = c[0:4] + c[4:8]
    c2 = c4[0:2] + c4[2:4]
    return c2[0:1] + c2[1:2]


def _bnlin_kernel(x_ref, wt_ref, b_ref, g_ref, be_ref, o_ref):
    x = x_ref[...].astype(jnp.bfloat16)
    y = jnp.dot(x, wt_ref[...].astype(jnp.bfloat16),
                preferred_element_type=jnp.float32) + b_ref[...]
    m = _batch_sum(y) * (1.0 / B)
    d = y - m
    v = _batch_sum(d * d) * (1.0 / B)
    o_ref[...] = jnp.maximum(
        d / jnp.sqrt(v + _EPS) * g_ref[...] + be_ref[...], 0.0)


def _mmT_kernel(hT_ref, a_ref, o_ref):
    o_ref[...] = jax.lax.dot_general(
        hT_ref[...].astype(jnp.bfloat16), a_ref[...].astype(jnp.bfloat16),
        (((1,), (1,)), ((), ())), preferred_element_type=jnp.float32)


@functools.partial(jax.jit)
def _run(text, image, in_aff, out_aff, Wc_t, b_c, g_c, be_c,
         W1_t, b1, g1, be1, W2_t, b2, g2, be2):
    BA = 256
    pool = pl.pallas_call(
        _pool_kernel,
        grid=(B // BA,),
        in_specs=[
            pl.BlockSpec((D, BA), lambda i: (0, i)),
            pl.BlockSpec((D, BA), lambda i: (0, i)),
        ],
        out_specs=pl.BlockSpec((D, BA), lambda i: (0, i)),
        out_shape=jax.ShapeDtypeStruct((D, B), jnp.float32),
    )
    p = pool(text.T, image.T).T

    bnlin = pl.pallas_call(
        _bnlin_kernel,
        out_shape=jax.ShapeDtypeStruct((B, D), jnp.float32),
    )

    BN = 512
    mm = pl.pallas_call(
        _mmT_kernel,
        grid=(B // BN,),
        in_specs=[
            pl.BlockSpec((D, B), lambda i: (0, 0)),
            pl.BlockSpec((BN, B), lambda i: (i, 0)),
        ],
        out_specs=pl.BlockSpec((D, BN), lambda i: (0, i)),
        out_shape=jax.ShapeDtypeStruct((D, B), jnp.float32),
    )

    h0 = bnlin(p, Wc_t, b_c, g_c, be_c)
    a1 = mm(h0.T, in_aff).T
    h1 = bnlin(a1, W1_t, b1, g1, be1)
    a2 = mm(h1.T, out_aff).T
    h2 = bnlin(a2, W2_t, b2, g2, be2)
    return h2


def kernel(text, image, in_aff, out_aff, W_c, b_c, g_c, be_c,
           W1, b1, g1, be1, W2, b2, g2, be2):
    r = lambda x: x.reshape(1, D)
    return _run(text, image, in_aff, out_aff,
                W_c.T, r(b_c), r(g_c), r(be_c),
                W1.T, r(b1), r(g1), r(be1),
                W2.T, r(b2), r(g2), r(be2))


# speedup vs baseline: 5.8730x; 1.0764x over previous
import functools
import math

import jax
import jax.numpy as jnp
from jax.experimental import pallas as pl

B = 4096
D = 64
_EPS = 1e-5


def _pool_kernel(t_ref, i_ref, p_ref):
    tT = t_ref[...].T * (1.0 / math.sqrt(D))
    iT = i_ref[...].T
    s = tT[:, None, :] * iT[None, :, :]
    m = jnp.max(s, axis=1, keepdims=True)
    e = jnp.exp(s - m)
    acc = e[:, 0:8, :]
    for tt in range(1, 8):
        acc = acc + e[:, 8 * tt:8 * tt + 8, :]
    a4 = acc[:, 0:4, :] + acc[:, 4:8, :]
    a2 = a4[:, 0:2, :] + a4[:, 2:4, :]
    denom = a2[:, 0, :] + a2[:, 1, :]
    p_ref[...] = (1.0 / denom).T


def _batch_sum(y):
    y4 = y.reshape(32, 16, 8, D)
    acc = y4[0]
    for mm in range(1, 32):
        acc = acc + y4[mm]
    c = acc[0]
    for u in range(1, 16):
        c = c + acc[u]
    c4 = c[0:4] + c[4:8]
    c2 = c4[0:2] + c4[2:4]
    return c2[0:1] + c2[1:2]


def _bnlin_body(x_ref, wt_ref, b_ref, g_ref, be_ref):
    x = x_ref[...].astype(jnp.bfloat16)
    y = jnp.dot(x, wt_ref[...].astype(jnp.bfloat16),
                preferred_element_type=jnp.float32) + b_ref[...]
    m = _batch_sum(y) * (1.0 / B)
    d = y - m
    v = _batch_sum(d * d) * (1.0 / B)
    return jnp.maximum(d / jnp.sqrt(v + _EPS) * g_ref[...] + be_ref[...], 0.0)


def _bnlin_kernel(x_ref, wt_ref, b_ref, g_ref, be_ref, o_ref):
    o_ref[...] = _bnlin_body(x_ref, wt_ref, b_ref, g_ref, be_ref)


def _bnlin_t_kernel(x_ref, wt_ref, b_ref, g_ref, be_ref, o_ref):
    o_ref[...] = _bnlin_body(x_ref, wt_ref, b_ref, g_ref, be_ref).T


def _mmT_kernel(hT_ref, a_ref, o_ref):
    res = jax.lax.dot_general(
        hT_ref[...].astype(jnp.bfloat16), a_ref[...].astype(jnp.bfloat16),
        (((1,), (1,)), ((), ())), preferred_element_type=jnp.float32)
    o_ref[...] = res.T


@functools.partial(jax.jit)
def _run(text, image, in_aff, out_aff, Wc_t, b_c, g_c, be_c,
         W1_t, b1, g1, be1, W2_t, b2, g2, be2):
    BA = 256
    pool = pl.pallas_call(
        _pool_kernel,
        grid=(B // BA,),
        in_specs=[
            pl.BlockSpec((BA, D), lambda i: (i, 0)),
            pl.BlockSpec((BA, D), lambda i: (i, 0)),
        ],
        out_specs=pl.BlockSpec((BA, D), lambda i: (i, 0)),
        out_shape=jax.ShapeDtypeStruct((B, D), jnp.float32),
    )
    p = pool(text, image)

    bnlin = pl.pallas_call(
        _bnlin_kernel,
        out_shape=jax.ShapeDtypeStruct((B, D), jnp.float32),
    )
    bnlin_t = pl.pallas_call(
        _bnlin_t_kernel,
        out_shape=jax.ShapeDtypeStruct((D, B), jnp.float32),
    )

    BN = 512
    mm = pl.pallas_call(
        _mmT_kernel,
        grid=(B // BN,),
        in_specs=[
            pl.BlockSpec((D, B), lambda i: (0, 0)),
            pl.BlockSpec((BN, B), lambda i: (i, 0)),
        ],
        out_specs=pl.BlockSpec((BN, D), lambda i: (i, 0)),
        out_shape=jax.ShapeDtypeStruct((B, D), jnp.float32),
    )

    h0T = bnlin_t(p, Wc_t, b_c, g_c, be_c)
    a1 = mm(h0T, in_aff)
    h1T = bnlin_t(a1, W1_t, b1, g1, be1)
    a2 = mm(h1T, out_aff)
    h2 = bnlin(a2, W2_t, b2, g2, be2)
    return h2


def kernel(text, image, in_aff, out_aff, W_c, b_c, g_c, be_c,
           W1, b1, g1, be1, W2, b2, g2, be2):
    r = lambda x: x.reshape(1, D)
    return _run(text, image, in_aff, out_aff,
                W_c.T, r(b_c), r(g_c), r(be_c),
                W1.T, r(b1), r(g1), r(be1),
                W2.T, r(b2), r(g2), r(be2))
